# baseline (speedup 1.0000x reference)
import functools
import jax
import jax.numpy as jnp
import numpy as np
from jax.experimental import pallas as pl
from jax.experimental.pallas import tpu as pltpu


def _sqdist(src, dst):
    return (jnp.sum(src ** 2, -1)[:, :, None] + jnp.sum(dst ** 2, -1)[:, None, :]
            - 2.0 * jnp.einsum('bnc,bmc->bnm', src, dst))


def _index_points(points, idx):
    B = points.shape[0]
    batch = jnp.arange(B).reshape((B,) + (1,) * (idx.ndim - 1))
    return points[batch, idx]


def _fps(xyz, npoint):
    B, N, _ = xyz.shape
    distance0 = jnp.full((B, N), 1e10, dtype=xyz.dtype)
    farthest0 = jnp.zeros((B,), dtype=jnp.int32)

    def body(carry, _):
        distance, farthest = carry
        centroid = xyz[jnp.arange(B), farthest]
        dist = jnp.sum((xyz - centroid[:, None, :]) ** 2, -1)
        distance = jnp.minimum(distance, dist)
        nxt = jnp.argmax(distance, axis=-1).astype(jnp.int32)
        return (distance, nxt), farthest

    _, idxs = jax.lax.scan(body, (distance0, farthest0), None, length=npoint)
    return jnp.transpose(idxs)


def _ball_query(radius, nsample, xyz, new_xyz):
    B, N, _ = xyz.shape
    S = new_xyz.shape[1]
    sqrdists = _sqdist(new_xyz, xyz)
    group_idx = jnp.broadcast_to(jnp.arange(N, dtype=jnp.int32), (B, S, N))
    group_idx = jnp.where(sqrdists > radius ** 2, N, group_idx)
    group_idx = jnp.sort(group_idx, axis=-1)[:, :, :nsample]
    group_first = group_idx[:, :, :1]
    group_idx = jnp.where(group_idx == N, jnp.broadcast_to(group_first, group_idx.shape), group_idx)
    return group_idx


def _mlp_max_body(x_ref, w0_ref, b0_ref, w1_ref, b1_ref, w2_ref, b2_ref, o_ref, *, K):
    x = x_ref[0]
    h = jnp.maximum(jnp.dot(x, w0_ref[...], preferred_element_type=jnp.float32) + b0_ref[...], 0.0)
    h = jnp.maximum(jnp.dot(h, w1_ref[...], preferred_element_type=jnp.float32) + b1_ref[...], 0.0)
    h = jnp.maximum(jnp.dot(h, w2_ref[...], preferred_element_type=jnp.float32) + b2_ref[...], 0.0)
    TSK, C = h.shape
    o_ref[0] = jnp.max(h.reshape(TSK // K, K, C), axis=1)


def _mlp_max(grouped, Ws, bs, TS):
    B, S, K, Cin = grouped.shape
    x = grouped.reshape(B, S * K, Cin)
    C2 = Ws[2].shape[1]
    wspecs = []
    args = []
    for W, b in zip(Ws, bs):
        args.append(W)
        args.append(b.reshape(1, -1))
        wspecs.append(pl.BlockSpec(W.shape, lambda b_, s_: (0, 0)))
        wspecs.append(pl.BlockSpec((1, b.shape[0]), lambda b_, s_: (0, 0)))
    out = pl.pallas_call(
        functools.partial(_mlp_max_body, K=K),
        grid=(B, S // TS),
        in_specs=[pl.BlockSpec((1, TS * K, Cin), lambda b_, s_: (b_, s_, 0))] + wspecs,
        out_specs=pl.BlockSpec((1, TS, C2), lambda b_, s_: (b_, s_, 0)),
        out_shape=jax.ShapeDtypeStruct((B, S, C2), jnp.float32),
    )(x, *args)
    return out


def _sa_layer(xyz, points, npoint, radius, nsample, Ws, bs, TS):
    fps_idx = _fps(xyz, npoint)
    new_xyz = _index_points(xyz, fps_idx)
    idx = _ball_query(radius, nsample, xyz, new_xyz)
    grouped_xyz = _index_points(xyz, idx) - new_xyz[:, :, None, :]
    if points is not None:
        grouped = jnp.concatenate([grouped_xyz, _index_points(points, idx)], axis=-1)
    else:
        grouped = grouped_xyz
    new_points = _mlp_max(grouped, Ws, bs, TS)
    return new_xyz, new_points


def kernel(input,
           W1_0, b1_0, W1_1, b1_1, W1_2, b1_2,
           W2_0, b2_0, W2_1, b2_1, W2_2, b2_2,
           W3_0, b3_0, W3_1, b3_1, W3_2, b3_2,
           W4_0, b4_0, W4_1, b4_1, W4_2, b4_2):
    kw = dict(locals())
    cfgs = [(2048, 0.2, 64, 32), (1024, 0.4, 32, 64), (512, 0.8, 16, 128), (256, 1.2, 16, 128)]
    xyz, f = input, None
    for li, (npoint, radius, nsample, TS) in enumerate(cfgs, start=1):
        Ws = [kw[f"W{li}_{mi}"] for mi in range(3)]
        bs = [kw[f"b{li}_{mi}"] for mi in range(3)]
        xyz, f = _sa_layer(xyz, f, npoint, radius, nsample, Ws, bs, TS)
    return (xyz, f)


# speedup vs baseline: 11.9452x; 11.9452x over previous
import functools
import jax
import jax.numpy as jnp
import numpy as np
from jax import lax
from jax.experimental import pallas as pl
from jax.experimental.pallas import tpu as pltpu
from jax.experimental.pallas import tpu_sc as plsc


def _fps_body(xs_ref, ys_ref, zs_ref, o_ref, *, S):
    B, N = xs_ref.shape
    xs, ys, zs = xs_ref[...], ys_ref[...], zs_ref[...]
    lanes = lax.broadcasted_iota(jnp.int32, (B, N), 1)

    def step(i, carry):
        dist, far = carry
        msk = lanes == far
        cx = jnp.sum(jnp.where(msk, xs, 0.0), -1, keepdims=True)
        cy = jnp.sum(jnp.where(msk, ys, 0.0), -1, keepdims=True)
        cz = jnp.sum(jnp.where(msk, zs, 0.0), -1, keepdims=True)
        o_ref[pl.ds(i, 1)] = jnp.concatenate([cx, cy, cz], -1)[None]
        dx, dy, dz = xs - cx, ys - cy, zs - cz
        d = (dx * dx + dy * dy) + dz * dz
        dist = jnp.minimum(dist, d)
        mx = jnp.max(dist, -1, keepdims=True)
        nxt = jnp.min(jnp.where(dist == mx, lanes, N), -1, keepdims=True)
        return dist, nxt

    dist0 = jnp.full((B, N), 1e10, dtype=jnp.float32)
    far0 = jnp.zeros((B, 1), dtype=jnp.int32)
    lax.fori_loop(0, S, step, (dist0, far0))


def _fps(xyz, S):
    B, N, _ = xyz.shape
    xs = xyz[:, :, 0]
    ys = xyz[:, :, 1]
    zs = xyz[:, :, 2]
    out = pl.pallas_call(
        functools.partial(_fps_body, S=S),
        in_specs=[pl.BlockSpec((B, N), lambda: (0, 0))] * 3,
        out_specs=pl.BlockSpec((S, B, 3), lambda: (0, 0, 0)),
        out_shape=jax.ShapeDtypeStruct((S, B, 3), jnp.float32),
    )(xs, ys, zs)
    return out.transpose(1, 0, 2)


def _ballq_body(q_ref, pT_ref, o_ref, *, K, r2, N):
    q = q_ref[0]
    pT = pT_ref[0]
    TS = q.shape[0]
    q2 = jnp.sum(q * q, -1, keepdims=True)
    p2 = jnp.sum(pT * pT, 0, keepdims=True)
    d = q2 + p2 - 2.0 * jnp.dot(q, pT, preferred_element_type=jnp.float32)
    lanes = lax.broadcasted_iota(jnp.int32, (TS, N), 1)
    val = jnp.where(d > r2, N, lanes)
    cols = []
    for _ in range(K):
        m = jnp.min(val, -1, keepdims=True)
        cols.append(m)
        val = jnp.where(val == m, N, val)
    idx = jnp.concatenate(cols, -1)
    first = idx[:, :1]
    idx = jnp.where(idx == N, first, idx)
    idx = jnp.minimum(idx, N - 1)
    b = pl.program_id(0)
    o_ref[0] = idx + b * N


def _ball_query(radius, K, xyzT, new_xyz, TS):
    B, _, N = xyzT.shape
    S = new_xyz.shape[1]
    return pl.pallas_call(
        functools.partial(_ballq_body, K=K, r2=radius * radius, N=N),
        grid=(B, S // TS),
        in_specs=[
            pl.BlockSpec((1, TS, 3), lambda b, s: (b, s, 0)),
            pl.BlockSpec((1, 3, N), lambda b, s: (b, 0, 0)),
        ],
        out_specs=pl.BlockSpec((1, TS, K), lambda b, s: (b, s, 0)),
        out_shape=jax.ShapeDtypeStruct((B, S, K), jnp.int32),
    )(new_xyz, xyzT)


_NC, _NS, _CH = 2, 16, 128


def _gather_rows(table, idx_flat):
    Rt, D = table.shape
    R = idx_flat.shape[0]
    NW = _NC * _NS
    b_per_w = R // NW
    nch = b_per_w // _CH
    mesh = plsc.VectorSubcoreMesh(core_axis_name="c", subcore_axis_name="s")

    @functools.partial(
        pl.kernel,
        mesh=mesh,
        out_type=jax.ShapeDtypeStruct((R, D), jnp.float32),
        scratch_types=[
            pltpu.VMEM((b_per_w,), jnp.int32),
            pltpu.VMEM((_CH, D), jnp.float32),
            pltpu.VMEM((_CH, D), jnp.float32),
            pltpu.SemaphoreType.DMA,
            pltpu.SemaphoreType.DMA,
        ],
    )
    def k(table_hbm, idx_hbm, out_hbm, idx_v, buf0, buf1, sem0, sem1):
        wid = lax.axis_index("s") * _NC + lax.axis_index("c")
        base = wid * b_per_w
        pltpu.sync_copy(idx_hbm.at[pl.ds(base, b_per_w)], idx_v)
        bufs = (buf0, buf1)
        sems = (sem0, sem1)

        def fire(c, buf, sem):
            return pltpu.async_copy(
                table_hbm.at[idx_v.at[pl.ds(c * _CH, _CH)]], buf, sem)

        fire(0, buf0, sem0).wait()

        def body(c, carry):
            nxt = lax.min(c + 1, nch - 1)
            for par in range(2):
                @pl.when((c + 1) % 2 == par)
                def _():
                    cp = fire(nxt, bufs[par], sems[par])
                    pltpu.sync_copy(bufs[1 - par],
                                    out_hbm.at[pl.ds(base + c * _CH, _CH)])
                    cp.wait()
            return carry

        lax.fori_loop(0, nch - 1, body, 0)
        pltpu.sync_copy(
            bufs[(nch - 1) % 2], out_hbm.at[pl.ds(base + (nch - 1) * _CH, _CH)])

    return k(table, idx_flat)


def _mlp_body(g_ref, q_ref, wx_ref, wf_ref, b0_ref, w1_ref, b1_ref,
              w2_ref, b2_ref, o_ref, *, K, Cf):
    g = g_ref[0]
    q = q_ref[0]
    x0 = g[:, Cf:Cf + 3] - q
    h = jnp.dot(x0, wx_ref[...], preferred_element_type=jnp.float32)
    if Cf:
        h = h + jnp.dot(g[:, :Cf], wf_ref[...], preferred_element_type=jnp.float32)
    h = jnp.maximum(h + b0_ref[...], 0.0)
    h = jnp.maximum(jnp.dot(h, w1_ref[...], preferred_element_type=jnp.float32)
                    + b1_ref[...], 0.0)
    h = jnp.maximum(jnp.dot(h, w2_ref[...], preferred_element_type=jnp.float32)
                    + b2_ref[...], 0.0)
    TSK, C2 = h.shape
    o_ref[0] = jnp.max(h.reshape(TSK // K, K, C2), 1)


def _mlp_max(g, q_exp, Ws, bs, K, TS, Cf):
    B, SK, D = g.shape
    S = SK // K
    W0, W1, W2 = Ws
    Wx = W0[:3]
    Wf = W0[3:] if Cf else W0[:3]
    C2 = W2.shape[1]
    wargs = [Wx, Wf, bs[0].reshape(1, -1), W1, bs[1].reshape(1, -1),
             W2, bs[2].reshape(1, -1)]
    wspecs = [pl.BlockSpec(a.shape, functools.partial(lambda n, b, s: (0,) * n, a.ndim))
              for a in wargs]
    return pl.pallas_call(
        functools.partial(_mlp_body, K=K, Cf=Cf),
        grid=(B, S // TS),
        in_specs=[pl.BlockSpec((1, TS * K, D), lambda b, s: (b, s, 0)),
                  pl.BlockSpec((1, TS * K, 3), lambda b, s: (b, s, 0))] + wspecs,
        out_specs=pl.BlockSpec((1, TS, C2), lambda b, s: (b, s, 0)),
        out_shape=jax.ShapeDtypeStruct((B, S, C2), jnp.float32),
    )(g, q_exp, *wargs)


_CFGS = [
    (2048, 0.2, 64, 128, 256, 64),
    (1024, 0.4, 32, 256, 256, 128),
    (512, 0.8, 16, 384, 512, 256),
    (256, 1.2, 16, 384, 256, 256),
]


def _sa_layer(xyz, points, cfg, Ws, bs):
    S, radius, K, D_pad, TS_sel, TS_mlp = cfg
    B, N, _ = xyz.shape
    new_xyz = _fps(xyz, S)
    xyzT = xyz.transpose(0, 2, 1)
    idx = _ball_query(radius, K, xyzT, new_xyz, TS_sel)

    if points is None:
        tab = xyz.reshape(B * N, 3)
        Cf = 0
    else:
        tab = jnp.concatenate([points, xyz], -1).reshape(B * N, -1)
        Cf = points.shape[-1]
    tab = jnp.pad(tab, ((0, 0), (0, D_pad - tab.shape[1])))
    g = _gather_rows(tab, idx.reshape(-1)).reshape(B, S * K, D_pad)

    q_exp = jnp.broadcast_to(new_xyz[:, :, None, :], (B, S, K, 3)).reshape(B, S * K, 3)
    new_points = _mlp_max(g, q_exp, Ws, bs, K, TS_mlp, Cf)
    return new_xyz, new_points


def kernel(input,
           W1_0, b1_0, W1_1, b1_1, W1_2, b1_2,
           W2_0, b2_0, W2_1, b2_1, W2_2, b2_2,
           W3_0, b3_0, W3_1, b3_1, W3_2, b3_2,
           W4_0, b4_0, W4_1, b4_1, W4_2, b4_2):
    kw = dict(locals())
    xyz, f = input, None
    for li, cfg in enumerate(_CFGS, start=1):
        Ws = [kw[f"W{li}_{mi}"] for mi in range(3)]
        bs = [kw[f"b{li}_{mi}"] for mi in range(3)]
        xyz, f = _sa_layer(xyz, f, cfg, Ws, bs)
    return (xyz, f)


# speedup vs baseline: 12.8786x; 1.0781x over previous
import functools
import jax
import jax.numpy as jnp
import numpy as np
from jax import lax
from jax.experimental import pallas as pl
from jax.experimental.pallas import tpu as pltpu
from jax.experimental.pallas import tpu_sc as plsc


def _fps_body(xs_ref, ys_ref, zs_ref, o_ref, *, S):
    B, N = xs_ref.shape
    xs, ys, zs = xs_ref[...], ys_ref[...], zs_ref[...]
    lanes = lax.broadcasted_iota(jnp.int32, (B, N), 1)

    def step(i, carry):
        dist, far = carry
        msk = lanes == far
        cx = jnp.sum(jnp.where(msk, xs, 0.0), -1, keepdims=True)
        cy = jnp.sum(jnp.where(msk, ys, 0.0), -1, keepdims=True)
        cz = jnp.sum(jnp.where(msk, zs, 0.0), -1, keepdims=True)
        o_ref[pl.ds(i, 1)] = jnp.concatenate([cx, cy, cz], -1)[None]
        dx, dy, dz = xs - cx, ys - cy, zs - cz
        d = (dx * dx + dy * dy) + dz * dz
        dist = jnp.minimum(dist, d)
        mx = jnp.max(dist, -1, keepdims=True)
        nxt = jnp.min(jnp.where(dist == mx, lanes, N), -1, keepdims=True)
        return dist, nxt

    dist0 = jnp.full((B, N), 1e10, dtype=jnp.float32)
    far0 = jnp.zeros((B, 1), dtype=jnp.int32)
    lax.fori_loop(0, S, step, (dist0, far0))


def _fps(xyz, S):
    B, N, _ = xyz.shape
    xs = xyz[:, :, 0]
    ys = xyz[:, :, 1]
    zs = xyz[:, :, 2]
    out = pl.pallas_call(
        functools.partial(_fps_body, S=S),
        in_specs=[pl.BlockSpec((B, N), lambda: (0, 0))] * 3,
        out_specs=pl.BlockSpec((S, B, 3), lambda: (0, 0, 0)),
        out_shape=jax.ShapeDtypeStruct((S, B, 3), jnp.float32),
    )(xs, ys, zs)
    return out.transpose(1, 0, 2)


def _ballq_body(q_ref, pT_ref, o_ref, *, K, r2, N):
    q = q_ref[0]
    pT = pT_ref[0]
    TS = q.shape[0]
    q2 = jnp.sum(q * q, -1, keepdims=True)
    p2 = jnp.sum(pT * pT, 0, keepdims=True)
    d = q2 + p2 - 2.0 * jnp.dot(q, pT, preferred_element_type=jnp.float32)
    lanes = lax.broadcasted_iota(jnp.int32, (TS, N), 1)
    val = jnp.where(d > r2, N, lanes)
    cols = []
    for _ in range(K):
        m = jnp.min(val, -1, keepdims=True)
        cols.append(m)
        val = jnp.where(val == m, N, val)
    idx = jnp.concatenate(cols, -1)
    first = idx[:, :1]
    idx = jnp.where(idx == N, first, idx)
    idx = jnp.minimum(idx, N - 1)
    b = pl.program_id(0)
    o_ref[0] = idx + b * N


def _ball_query(radius, K, xyzT, new_xyz, TS):
    B, _, N = xyzT.shape
    S = new_xyz.shape[1]
    return pl.pallas_call(
        functools.partial(_ballq_body, K=K, r2=radius * radius, N=N),
        grid=(B, S // TS),
        in_specs=[
            pl.BlockSpec((1, TS, 3), lambda b, s: (b, s, 0)),
            pl.BlockSpec((1, 3, N), lambda b, s: (b, 0, 0)),
        ],
        out_specs=pl.BlockSpec((1, TS, K), lambda b, s: (b, s, 0)),
        out_shape=jax.ShapeDtypeStruct((B, S, K), jnp.int32),
    )(new_xyz, xyzT)


_NC, _NS, _CH = 2, 16, 128


def _gather_rows(table, idx_flat):
    Rt, D = table.shape
    R = idx_flat.shape[0]
    NW = _NC * _NS
    b_per_w = R // NW
    nch = b_per_w // _CH
    nbuf = 4 if (D <= 128 and nch >= 4) else 2
    mesh = plsc.VectorSubcoreMesh(core_axis_name="c", subcore_axis_name="s")

    @functools.partial(
        pl.kernel,
        mesh=mesh,
        out_type=jax.ShapeDtypeStruct((R, D), jnp.float32),
        scratch_types=[
            pltpu.VMEM((b_per_w,), jnp.int32),
        ] + [pltpu.VMEM((_CH, D), jnp.float32)] * nbuf
          + [pltpu.SemaphoreType.DMA] * nbuf,
    )
    def k(table_hbm, idx_hbm, out_hbm, idx_v, *bufsem):
        bufs = bufsem[:nbuf]
        sems = bufsem[nbuf:]
        wid = lax.axis_index("s") * _NC + lax.axis_index("c")
        base = wid * b_per_w
        pltpu.sync_copy(idx_hbm.at[pl.ds(base, b_per_w)], idx_v)

        def dma(c, i):
            return pltpu.make_async_copy(
                table_hbm.at[idx_v.at[pl.ds(c * _CH, _CH)]], bufs[i], sems[i])

        for j in range(nbuf - 1):
            dma(j, j).start()

        def body(c, carry):
            for i in range(nbuf):
                @pl.when(c % nbuf == i)
                def _():
                    @pl.when(c + nbuf - 1 < nch)
                    def _():
                        dma(c + nbuf - 1, (i + nbuf - 1) % nbuf).start()
                    dma(c, i).wait()
                    pltpu.sync_copy(bufs[i],
                                    out_hbm.at[pl.ds(base + c * _CH, _CH)])
            return carry

        lax.fori_loop(0, nch, body, 0)

    return k(table, idx_flat)


def _mlp_body(g_ref, q_ref, wx_ref, wf_ref, b0_ref, w1_ref, b1_ref,
              w2_ref, b2_ref, o_ref, *, K, Cf):
    g = g_ref[0]
    q = q_ref[0]
    x0 = g[:, Cf:Cf + 3] - q
    h = jnp.dot(x0, wx_ref[...], preferred_element_type=jnp.float32)
    if Cf:
        h = h + jnp.dot(g[:, :Cf], wf_ref[...], preferred_element_type=jnp.float32)
    h = jnp.maximum(h + b0_ref[...], 0.0)
    h = jnp.maximum(jnp.dot(h, w1_ref[...], preferred_element_type=jnp.float32)
                    + b1_ref[...], 0.0)
    h = jnp.maximum(jnp.dot(h, w2_ref[...], preferred_element_type=jnp.float32)
                    + b2_ref[...], 0.0)
    TSK, C2 = h.shape
    o_ref[0] = jnp.max(h.reshape(TSK // K, K, C2), 1)


def _mlp_max(g, q_exp, Ws, bs, K, TS, Cf):
    B, SK, D = g.shape
    S = SK // K
    W0, W1, W2 = Ws
    Wx = W0[:3]
    Wf = W0[3:] if Cf else W0[:3]
    C2 = W2.shape[1]
    wargs = [Wx, Wf, bs[0].reshape(1, -1), W1, bs[1].reshape(1, -1),
             W2, bs[2].reshape(1, -1)]
    wspecs = [pl.BlockSpec(a.shape, functools.partial(lambda n, b, s: (0,) * n, a.ndim))
              for a in wargs]
    return pl.pallas_call(
        functools.partial(_mlp_body, K=K, Cf=Cf),
        grid=(B, S // TS),
        in_specs=[pl.BlockSpec((1, TS * K, D), lambda b, s: (b, s, 0)),
                  pl.BlockSpec((1, TS * K, 3), lambda b, s: (b, s, 0))] + wspecs,
        out_specs=pl.BlockSpec((1, TS, C2), lambda b, s: (b, s, 0)),
        out_shape=jax.ShapeDtypeStruct((B, S, C2), jnp.float32),
    )(g, q_exp, *wargs)


_CFGS = [
    (2048, 0.2, 64, 128, 256, 64),
    (1024, 0.4, 32, 256, 256, 128),
    (512, 0.8, 16, 384, 512, 256),
    (256, 1.2, 16, 384, 256, 256),
]


def _sa_layer(xyz, points, cfg, Ws, bs):
    S, radius, K, D_pad, TS_sel, TS_mlp = cfg
    B, N, _ = xyz.shape
    new_xyz = _fps(xyz, S)
    xyzT = xyz.transpose(0, 2, 1)
    idx = _ball_query(radius, K, xyzT, new_xyz, TS_sel)

    if points is None:
        tab = xyz.reshape(B * N, 3)
        Cf = 0
    else:
        tab = jnp.concatenate([points, xyz], -1).reshape(B * N, -1)
        Cf = points.shape[-1]
    tab = jnp.pad(tab, ((0, 0), (0, D_pad - tab.shape[1])))
    g = _gather_rows(tab, idx.reshape(-1)).reshape(B, S * K, D_pad)

    q_exp = jnp.broadcast_to(new_xyz[:, :, None, :], (B, S, K, 3)).reshape(B, S * K, 3)
    new_points = _mlp_max(g, q_exp, Ws, bs, K, TS_mlp, Cf)
    return new_xyz, new_points


def kernel(input,
           W1_0, b1_0, W1_1, b1_1, W1_2, b1_2,
           W2_0, b2_0, W2_1, b2_1, W2_2, b2_2,
           W3_0, b3_0, W3_1, b3_1, W3_2, b3_2,
           W4_0, b4_0, W4_1, b4_1, W4_2, b4_2):
    kw = dict(locals())
    xyz, f = input, None
    for li, cfg in enumerate(_CFGS, start=1):
        Ws = [kw[f"W{li}_{mi}"] for mi in range(3)]
        bs = [kw[f"b{li}_{mi}"] for mi in range(3)]
        xyz, f = _sa_layer(xyz, f, cfg, Ws, bs)
    return (xyz, f)


# speedup vs baseline: 12.8885x; 1.0008x over previous
import functools
import jax
import jax.numpy as jnp
import numpy as np
from jax import lax
from jax.experimental import pallas as pl
from jax.experimental.pallas import tpu as pltpu
from jax.experimental.pallas import tpu_sc as plsc


def _fps_body(xs_ref, ys_ref, zs_ref, o_ref, *, S):
    B, N = xs_ref.shape
    xs, ys, zs = xs_ref[...], ys_ref[...], zs_ref[...]
    lanes = lax.broadcasted_iota(jnp.int32, (B, N), 1)

    def step(i, carry):
        dist, far = carry
        msk = lanes == far
        cx = jnp.sum(jnp.where(msk, xs, 0.0), -1, keepdims=True)
        cy = jnp.sum(jnp.where(msk, ys, 0.0), -1, keepdims=True)
        cz = jnp.sum(jnp.where(msk, zs, 0.0), -1, keepdims=True)
        o_ref[pl.ds(i, 1)] = jnp.concatenate([cx, cy, cz], -1)[None]
        dx, dy, dz = xs - cx, ys - cy, zs - cz
        d = (dx * dx + dy * dy) + dz * dz
        dist = jnp.minimum(dist, d)
        mx = jnp.max(dist, -1, keepdims=True)
        nxt = jnp.min(jnp.where(dist == mx, lanes, N), -1, keepdims=True)
        return dist, nxt

    dist0 = jnp.full((B, N), 1e10, dtype=jnp.float32)
    far0 = jnp.zeros((B, 1), dtype=jnp.int32)
    lax.fori_loop(0, S, step, (dist0, far0))


def _fps(xyz, S):
    B, N, _ = xyz.shape
    xs = xyz[:, :, 0]
    ys = xyz[:, :, 1]
    zs = xyz[:, :, 2]
    out = pl.pallas_call(
        functools.partial(_fps_body, S=S),
        in_specs=[pl.BlockSpec((B, N), lambda: (0, 0))] * 3,
        out_specs=pl.BlockSpec((S, B, 3), lambda: (0, 0, 0)),
        out_shape=jax.ShapeDtypeStruct((S, B, 3), jnp.float32),
    )(xs, ys, zs)
    return out.transpose(1, 0, 2)


def _ballq_body(q_ref, pT_ref, o_ref, *, K, r2, N):
    q = q_ref[0]
    pT = pT_ref[0]
    TS = q.shape[0]
    q2 = jnp.sum(q * q, -1, keepdims=True)
    p2 = jnp.sum(pT * pT, 0, keepdims=True)
    d = q2 + p2 - 2.0 * jnp.dot(q, pT, preferred_element_type=jnp.float32)
    lanes = lax.broadcasted_iota(jnp.int32, (TS, N), 1)
    val = jnp.where(d > r2, N, lanes)
    cols = []
    for _ in range(K):
        m = jnp.min(val, -1, keepdims=True)
        cols.append(m)
        val = jnp.where(val == m, N, val)
    idx = jnp.concatenate(cols, -1)
    first = idx[:, :1]
    idx = jnp.where(idx == N, first, idx)
    idx = jnp.minimum(idx, N - 1)
    b = pl.program_id(0)
    o_ref[0] = idx + b * N


def _ball_query(radius, K, xyzT, new_xyz, TS):
    B, _, N = xyzT.shape
    S = new_xyz.shape[1]
    return pl.pallas_call(
        functools.partial(_ballq_body, K=K, r2=radius * radius, N=N),
        grid=(B, S // TS),
        in_specs=[
            pl.BlockSpec((1, TS, 3), lambda b, s: (b, s, 0)),
            pl.BlockSpec((1, 3, N), lambda b, s: (b, 0, 0)),
        ],
        out_specs=pl.BlockSpec((1, TS, K), lambda b, s: (b, s, 0)),
        out_shape=jax.ShapeDtypeStruct((B, S, K), jnp.int32),
    )(new_xyz, xyzT)


_NC, _NS, _CH = 2, 16, 128


def _gather_rows(table, idx_flat):
    Rt, D = table.shape
    R = idx_flat.shape[0]
    NW = _NC * _NS
    b_per_w = R // NW
    nch = b_per_w // _CH
    if D <= 128 and nch >= 4:
        nbuf = 4
    elif D <= 256 and nch >= 3:
        nbuf = 3
    else:
        nbuf = 2
    mesh = plsc.VectorSubcoreMesh(core_axis_name="c", subcore_axis_name="s")

    @functools.partial(
        pl.kernel,
        mesh=mesh,
        out_type=jax.ShapeDtypeStruct((R, D), jnp.float32),
        scratch_types=[
            pltpu.VMEM((b_per_w,), jnp.int32),
        ] + [pltpu.VMEM((_CH, D), jnp.float32)] * nbuf
          + [pltpu.SemaphoreType.DMA] * nbuf,
    )
    def k(table_hbm, idx_hbm, out_hbm, idx_v, *bufsem):
        bufs = bufsem[:nbuf]
        sems = bufsem[nbuf:]
        wid = lax.axis_index("s") * _NC + lax.axis_index("c")
        base = wid * b_per_w
        pltpu.sync_copy(idx_hbm.at[pl.ds(base, b_per_w)], idx_v)

        def dma(c, i):
            return pltpu.make_async_copy(
                table_hbm.at[idx_v.at[pl.ds(c * _CH, _CH)]], bufs[i], sems[i])

        for j in range(nbuf - 1):
            dma(j, j).start()

        def body(c, carry):
            for i in range(nbuf):
                @pl.when(c % nbuf == i)
                def _():
                    @pl.when(c + nbuf - 1 < nch)
                    def _():
                        dma(c + nbuf - 1, (i + nbuf - 1) % nbuf).start()
                    dma(c, i).wait()
                    pltpu.sync_copy(bufs[i],
                                    out_hbm.at[pl.ds(base + c * _CH, _CH)])
            return carry

        lax.fori_loop(0, nch, body, 0)

    return k(table, idx_flat)


def _mlp_body(g_ref, q_ref, wx_ref, wf_ref, b0_ref, w1_ref, b1_ref,
              w2_ref, b2_ref, o_ref, *, K, Cf):
    g = g_ref[0]
    q = q_ref[0]
    x0 = g[:, Cf:Cf + 3] - q
    h = jnp.dot(x0, wx_ref[...], preferred_element_type=jnp.float32)
    if Cf:
        h = h + jnp.dot(g[:, :Cf], wf_ref[...], preferred_element_type=jnp.float32)
    h = jnp.maximum(h + b0_ref[...], 0.0)
    h = jnp.maximum(jnp.dot(h, w1_ref[...], preferred_element_type=jnp.float32)
                    + b1_ref[...], 0.0)
    h = jnp.maximum(jnp.dot(h, w2_ref[...], preferred_element_type=jnp.float32)
                    + b2_ref[...], 0.0)
    TSK, C2 = h.shape
    o_ref[0] = jnp.max(h.reshape(TSK // K, K, C2), 1)


def _mlp_max(g, q_exp, Ws, bs, K, TS, Cf):
    B, SK, D = g.shape
    S = SK // K
    W0, W1, W2 = Ws
    Wx = W0[:3]
    Wf = W0[3:] if Cf else W0[:3]
    C2 = W2.shape[1]
    wargs = [Wx, Wf, bs[0].reshape(1, -1), W1, bs[1].reshape(1, -1),
             W2, bs[2].reshape(1, -1)]
    wspecs = [pl.BlockSpec(a.shape, functools.partial(lambda n, b, s: (0,) * n, a.ndim))
              for a in wargs]
    return pl.pallas_call(
        functools.partial(_mlp_body, K=K, Cf=Cf),
        grid=(B, S // TS),
        in_specs=[pl.BlockSpec((1, TS * K, D), lambda b, s: (b, s, 0)),
                  pl.BlockSpec((1, TS * K, 3), lambda b, s: (b, s, 0))] + wspecs,
        out_specs=pl.BlockSpec((1, TS, C2), lambda b, s: (b, s, 0)),
        out_shape=jax.ShapeDtypeStruct((B, S, C2), jnp.float32),
    )(g, q_exp, *wargs)


_CFGS = [
    (2048, 0.2, 64, 128, 256, 64),
    (1024, 0.4, 32, 256, 256, 128),
    (512, 0.8, 16, 384, 512, 256),
    (256, 1.2, 16, 384, 256, 256),
]


def _sa_layer(xyz, points, cfg, Ws, bs):
    S, radius, K, D_pad, TS_sel, TS_mlp = cfg
    B, N, _ = xyz.shape
    new_xyz = _fps(xyz, S)
    xyzT = xyz.transpose(0, 2, 1)
    idx = _ball_query(radius, K, xyzT, new_xyz, TS_sel)

    if points is None:
        tab = xyz.reshape(B * N, 3)
        Cf = 0
    else:
        tab = jnp.concatenate([points, xyz], -1).reshape(B * N, -1)
        Cf = points.shape[-1]
    tab = jnp.pad(tab, ((0, 0), (0, D_pad - tab.shape[1])))
    g = _gather_rows(tab, idx.reshape(-1)).reshape(B, S * K, D_pad)

    q_exp = jnp.broadcast_to(new_xyz[:, :, None, :], (B, S, K, 3)).reshape(B, S * K, 3)
    new_points = _mlp_max(g, q_exp, Ws, bs, K, TS_mlp, Cf)
    return new_xyz, new_points


def kernel(input,
           W1_0, b1_0, W1_1, b1_1, W1_2, b1_2,
           W2_0, b2_0, W2_1, b2_1, W2_2, b2_2,
           W3_0, b3_0, W3_1, b3_1, W3_2, b3_2,
           W4_0, b4_0, W4_1, b4_1, W4_2, b4_2):
    kw = dict(locals())
    xyz, f = input, None
    for li, cfg in enumerate(_CFGS, start=1):
        Ws = [kw[f"W{li}_{mi}"] for mi in range(3)]
        bs = [kw[f"b{li}_{mi}"] for mi in range(3)]
        xyz, f = _sa_layer(xyz, f, cfg, Ws, bs)
    return (xyz, f)
